# baseline (speedup 1.0000x reference)
import functools

import jax
import jax.numpy as jnp
from jax.experimental import pallas as pl
from jax.experimental.pallas import tpu as pltpu

_ALPHA = 3.0
_K = 20
_ROW_BLOCK = 128
_CHUNK = 128


def _nodevec_kernel(e1_ref, w1_ref, b1_ref, e2_ref, w2_ref, b2_ref,
                    nv1_ref, nv2_ref):
    n = e1_ref.shape[0]
    h1 = jax.lax.dot_general(
        e1_ref[...], w1_ref[...], (((1,), (1,)), ((), ())),
        preferred_element_type=jnp.float32)
    h2 = jax.lax.dot_general(
        e2_ref[...], w2_ref[...], (((1,), (1,)), ((), ())),
        preferred_element_type=jnp.float32)
    nv1_ref[:n, :] = jnp.tanh(_ALPHA * (h1 + b1_ref[...]))
    nv2_ref[:n, :] = jnp.tanh(_ALPHA * (h2 + b2_ref[...]))
    nv1_ref[n:, :] = jnp.zeros_like(nv1_ref[n:, :])
    nv2_ref[n:, :] = jnp.zeros_like(nv2_ref[n:, :])


def _adj_kernel(nv1b_ref, nv2b_ref, nv1_ref, nv2_ref, expand_ref, tri_ref,
                cmat_ref, out_ref, *, n):
    r, np_ = nv1b_ref.shape[0], nv1_ref.shape[0]
    nc = np_ // _CHUNK
    a = jax.lax.dot_general(
        nv1b_ref[...], nv2_ref[...], (((1,), (1,)), ((), ())),
        preferred_element_type=jnp.float32)
    a -= jax.lax.dot_general(
        nv2b_ref[...], nv1_ref[...], (((1,), (1,)), ((), ())),
        preferred_element_type=jnp.float32)
    t = jnp.tanh(_ALPHA * a)
    eqf = (t >= 1.0).astype(jnp.bfloat16)

    expand = expand_ref[...]
    cmat = cmat_ref[...]
    s = jax.lax.dot_general(
        eqf, expand, (((1,), (0,)), ((), ())),
        preferred_element_type=jnp.float32)
    p = jax.lax.dot_general(
        s, tri_ref[...], (((1,), (0,)), ((), ())),
        preferred_element_type=jnp.float32)

    row0 = pl.program_id(0) * r
    rowid = row0 + jax.lax.broadcasted_iota(jnp.int32, (r, 1), 0)
    cnt = jnp.where(rowid < n, p[:, nc - 1:nc], jnp.inf)
    fast = jnp.min(cnt) >= _K

    @pl.when(fast)
    def _fast_path():
        pprev = p - s
        fk = (p <= _K).astype(jnp.bfloat16)
        bnd = ((pprev < _K) & (p > _K)).astype(jnp.bfloat16)
        fk_l = jax.lax.dot_general(
            fk, expand, (((1,), (1,)), ((), ())),
            preferred_element_type=jnp.float32)
        bnd_l = jax.lax.dot_general(
            bnd, expand, (((1,), (1,)), ((), ())),
            preferred_element_type=jnp.float32)
        masked = eqf * bnd_l.astype(jnp.bfloat16)
        eqb = jax.lax.dot_general(
            masked, cmat, (((1,), (0,)), ((), ())),
            preferred_element_type=jnp.float32)
        lane = jax.lax.broadcasted_iota(jnp.int32, (r, _CHUNK), 1)
        w = eqb
        shift = 1
        while shift < _CHUNK:
            w = w + jnp.where(lane >= shift,
                              pltpu.roll(w, shift, axis=1), 0.0)
            shift *= 2
        need = _K - jnp.sum(pprev * bnd.astype(jnp.float32),
                            axis=1, keepdims=True)
        lk = (w <= need).astype(jnp.bfloat16)
        lk_l = jax.lax.dot_general(
            lk, cmat, (((1,), (1,)), ((), ())),
            preferred_element_type=jnp.float32)
        out_ref[...] = (eqf * fk_l + masked * lk_l)[:, :n]

    @pl.when(jnp.logical_not(fast))
    def _general_path():
        iota = jax.lax.broadcasted_iota(jnp.int32, (r, np_), 1)
        adj0 = jnp.maximum(t, 0.0)

        def body(_, carry):
            work, keep = carry
            m = jnp.max(work, axis=1, keepdims=True)
            cand = jnp.where(work == m, iota, np_)
            j = jnp.min(cand, axis=1, keepdims=True)
            sel = iota == j
            keep = jnp.where(sel & (m > 0.0), 1.0, keep)
            work = jnp.where(sel, -1.0, work)
            return work, keep

        _, keep = jax.lax.fori_loop(
            0, _K, body, (adj0, jnp.zeros((r, np_), jnp.float32)))
        out_ref[...] = (adj0 * keep)[:, :n]


def kernel(idx, scale_idx, scale_set, emb1, emb2, W1, b1, W2, b2):
    del scale_idx, scale_set
    e1 = jnp.take(emb1, idx, axis=0)
    e2 = jnp.take(emb2, idx, axis=0)
    n, d = e1.shape
    np_ = (n + 1023) // 1024 * 1024

    nv1, nv2 = pl.pallas_call(
        _nodevec_kernel,
        out_shape=(jax.ShapeDtypeStruct((np_, d), jnp.float32),
                   jax.ShapeDtypeStruct((np_, d), jnp.float32)),
    )(e1, W1, b1.reshape(1, d), e2, W2, b2.reshape(1, d))

    nc = np_ // _CHUNK
    g = jnp.arange(np_, dtype=jnp.int32)
    expand = (g[:, None] // _CHUNK == jnp.arange(nc)[None, :]
              ).astype(jnp.bfloat16)
    tri = (jnp.arange(nc)[:, None] <= jnp.arange(nc)[None, :]
           ).astype(jnp.float32)
    cmat = (g[:, None] % _CHUNK == jnp.arange(_CHUNK)[None, :]
            ).astype(jnp.bfloat16)

    rb = _ROW_BLOCK
    grid = (n + rb - 1) // rb
    adj = pl.pallas_call(
        functools.partial(_adj_kernel, n=n),
        grid=(grid,),
        in_specs=[
            pl.BlockSpec((rb, d), lambda i: (i, 0)),
            pl.BlockSpec((rb, d), lambda i: (i, 0)),
            pl.BlockSpec((np_, d), lambda i: (0, 0)),
            pl.BlockSpec((np_, d), lambda i: (0, 0)),
            pl.BlockSpec((np_, nc), lambda i: (0, 0)),
            pl.BlockSpec((nc, nc), lambda i: (0, 0)),
            pl.BlockSpec((np_, _CHUNK), lambda i: (0, 0)),
        ],
        out_specs=pl.BlockSpec((rb, n), lambda i: (i, 0)),
        out_shape=jax.ShapeDtypeStruct((n, n), jnp.float32),
        compiler_params=pltpu.CompilerParams(
            vmem_limit_bytes=100 * 1024 * 1024),
    )(nv1, nv2, nv1, nv2, expand, tri, cmat)
    return adj


# speedup vs baseline: 1.1692x; 1.1692x over previous
import functools

import jax
import jax.numpy as jnp
from jax.experimental import pallas as pl
from jax.experimental.pallas import tpu as pltpu

_ALPHA = 3.0
_K = 20
_ROW_BLOCK = 128
_CHUNK = 128


def _nodevec_kernel(e1_ref, w1_ref, b1_ref, e2_ref, w2_ref, b2_ref,
                    acat_ref, bcat_ref):
    n, d = e1_ref.shape
    h1 = jax.lax.dot_general(
        e1_ref[...], w1_ref[...], (((1,), (1,)), ((), ())),
        preferred_element_type=jnp.float32)
    h2 = jax.lax.dot_general(
        e2_ref[...], w2_ref[...], (((1,), (1,)), ((), ())),
        preferred_element_type=jnp.float32)
    nv1 = jnp.tanh(_ALPHA * (h1 + b1_ref[...]))
    nv2 = jnp.tanh(_ALPHA * (h2 + b2_ref[...]))
    acat_ref[:n, :d] = nv1
    acat_ref[:n, d:] = -nv2
    bcat_ref[:n, :d] = nv2
    bcat_ref[:n, d:] = nv1
    acat_ref[n:, :] = jnp.zeros_like(acat_ref[n:, :])
    bcat_ref[n:, :] = jnp.zeros_like(bcat_ref[n:, :])


def _adj_kernel(acat_ref, bcat_ref, expand_ref, tri_ref,
                cmat_ref, out_ref, *, n):
    r, np_ = acat_ref.shape[0], bcat_ref.shape[0]
    nc = np_ // _CHUNK
    a = jax.lax.dot_general(
        acat_ref[...], bcat_ref[...], (((1,), (1,)), ((), ())),
        preferred_element_type=jnp.float32)
    t = jnp.tanh(_ALPHA * a)
    eqf = (t >= 1.0).astype(jnp.bfloat16)

    expand = expand_ref[...]
    cmat = cmat_ref[...]
    s = jax.lax.dot_general(
        eqf, expand, (((1,), (0,)), ((), ())),
        preferred_element_type=jnp.float32)
    p = jax.lax.dot_general(
        s, tri_ref[...], (((1,), (0,)), ((), ())),
        preferred_element_type=jnp.float32)

    row0 = pl.program_id(0) * r
    rowid = row0 + jax.lax.broadcasted_iota(jnp.int32, (r, 1), 0)
    cnt = jnp.where(rowid < n, p[:, nc - 1:nc], jnp.inf)
    fast = jnp.min(cnt) >= _K

    @pl.when(fast)
    def _fast_path():
        pprev = p - s
        fk = (p <= _K).astype(jnp.float32)
        bnd = ((pprev < _K) & (p > _K)).astype(jnp.float32)
        code = (fk + 2.0 * bnd).astype(jnp.bfloat16)
        code_l = jax.lax.dot_general(
            code, expand, (((1,), (1,)), ((), ())),
            preferred_element_type=jnp.float32)
        masked = jnp.where(code_l > 1.5, eqf, jnp.bfloat16(0))
        eqb = jax.lax.dot_general(
            masked, cmat, (((1,), (0,)), ((), ())),
            preferred_element_type=jnp.float32)
        lane = jax.lax.broadcasted_iota(jnp.int32, (r, _CHUNK), 1)
        w = eqb
        shift = 1
        while shift < _CHUNK:
            w = w + jnp.where(lane >= shift,
                              pltpu.roll(w, shift, axis=1), 0.0)
            shift *= 2
        need = _K - jnp.sum(bnd * pprev, axis=1, keepdims=True)
        lk = (w <= need).astype(jnp.bfloat16)
        lk_l = jax.lax.dot_general(
            lk, cmat, (((1,), (1,)), ((), ())),
            preferred_element_type=jnp.float32)
        out_ref[...] = jnp.where(code_l == 1.0,
                                 eqf.astype(jnp.float32),
                                 masked * lk_l)[:, :n]

    @pl.when(jnp.logical_not(fast))
    def _general_path():
        iota = jax.lax.broadcasted_iota(jnp.int32, (r, np_), 1)
        adj0 = jnp.maximum(t, 0.0)

        def body(_, carry):
            work, keep = carry
            m = jnp.max(work, axis=1, keepdims=True)
            cand = jnp.where(work == m, iota, np_)
            j = jnp.min(cand, axis=1, keepdims=True)
            sel = iota == j
            keep = jnp.where(sel & (m > 0.0), 1.0, keep)
            work = jnp.where(sel, -1.0, work)
            return work, keep

        _, keep = jax.lax.fori_loop(
            0, _K, body, (adj0, jnp.zeros((r, np_), jnp.float32)))
        out_ref[...] = (adj0 * keep)[:, :n]


def kernel(idx, scale_idx, scale_set, emb1, emb2, W1, b1, W2, b2):
    del scale_idx, scale_set
    e1 = jnp.take(emb1, idx, axis=0)
    e2 = jnp.take(emb2, idx, axis=0)
    n, d = e1.shape
    np_ = (n + 1023) // 1024 * 1024

    acat, bcat = pl.pallas_call(
        _nodevec_kernel,
        out_shape=(jax.ShapeDtypeStruct((np_, 2 * d), jnp.float32),
                   jax.ShapeDtypeStruct((np_, 2 * d), jnp.float32)),
    )(e1, W1, b1.reshape(1, d), e2, W2, b2.reshape(1, d))

    nc = np_ // _CHUNK
    g = jnp.arange(np_, dtype=jnp.int32)
    expand = (g[:, None] // _CHUNK == jnp.arange(nc)[None, :]
              ).astype(jnp.bfloat16)
    tri = (jnp.arange(nc)[:, None] <= jnp.arange(nc)[None, :]
           ).astype(jnp.float32)
    cmat = (g[:, None] % _CHUNK == jnp.arange(_CHUNK)[None, :]
            ).astype(jnp.bfloat16)

    rb = _ROW_BLOCK
    grid = (n + rb - 1) // rb
    adj = pl.pallas_call(
        functools.partial(_adj_kernel, n=n),
        grid=(grid,),
        in_specs=[
            pl.BlockSpec((rb, 2 * d), lambda i: (i, 0)),
            pl.BlockSpec((np_, 2 * d), lambda i: (0, 0)),
            pl.BlockSpec((np_, nc), lambda i: (0, 0)),
            pl.BlockSpec((nc, nc), lambda i: (0, 0)),
            pl.BlockSpec((np_, _CHUNK), lambda i: (0, 0)),
        ],
        out_specs=pl.BlockSpec((rb, n), lambda i: (i, 0)),
        out_shape=jax.ShapeDtypeStruct((n, n), jnp.float32),
        compiler_params=pltpu.CompilerParams(
            vmem_limit_bytes=100 * 1024 * 1024),
    )(acat, bcat, expand, tri, cmat)
    return adj
